# baseline (speedup 1.0000x reference)
import functools

import jax
import jax.numpy as jnp
from jax import lax
from jax.experimental import pallas as pl
from jax.experimental.pallas import tpu as pltpu
from jax.experimental.pallas import tpu_sc as plsc

_B = 128
_V = 100000
_TAIL = 32
_VMAIN = _V - _TAIL
_NS = 16
_BANDS = 8
_NBUF = 2
_BIG = 2**30
_W0 = 48 * 128
_NVEC = _W0 // 16


def _perm(x, idx):
    dnums = lax.GatherDimensionNumbers(
        offset_dims=(), collapsed_slice_dims=(0,), start_index_map=(0,))
    return lax.gather(x, idx[:, None], dnums, (1,),
                      mode=lax.GatherScatterMode.PROMISE_IN_BOUNDS)


def _allmax(x, lane):
    for sh in (8, 4, 2, 1):
        x = jnp.maximum(x, _perm(x, lane ^ sh))
    return x


def _allmin(x, lane):
    for sh in (8, 4, 2, 1):
        x = jnp.minimum(x, _perm(x, lane ^ sh))
    return x

def _sc_body(logits, ids_out, probs_out, buf, tbuf, valc, colc, idsv,
             sval, scol, mval, mcol, sems_in, sems_out, sems_t):
    c = lax.axis_index("c")
    s = lax.axis_index("s")
    lane = lax.iota(jnp.int32, 16)
    start = s * _W0 + jnp.minimum(s, 13) * 128
    extra = s < 13
    tail_here = s == _NS - 1

    def rows0(b):
        return c * 64 + b * 8

    def in_cp(b, slot):
        return pltpu.make_async_copy(
            logits.at[pl.ds(rows0(b), 8), pl.ds(start, _W0)],
            buf.at[slot, :, pl.ds(0, _W0)], sems_in.at[slot])

    def in_cp_x(b, slot):
        return pltpu.make_async_copy(
            logits.at[pl.ds(rows0(b), 8), pl.ds(start + _W0, 128)],
            buf.at[slot, :, pl.ds(_W0, 128)], sems_in.at[slot])

    def out_cp(b, slot):
        return pltpu.make_async_copy(
            buf.at[slot, :, pl.ds(0, _W0)],
            probs_out.at[pl.ds(rows0(b), 8), pl.ds(start, _W0)],
            sems_out.at[slot])

    def out_cp_x(b, slot):
        return pltpu.make_async_copy(
            buf.at[slot, :, pl.ds(_W0, 128)],
            probs_out.at[pl.ds(rows0(b), 8), pl.ds(start + _W0, 128)],
            sems_out.at[slot])

    def t_in(b):
        return pltpu.make_async_copy(
            logits.at[pl.ds(rows0(b), 8), pl.ds(_VMAIN, _TAIL)],
            tbuf.at[b], sems_t)

    def t_out(b):
        return pltpu.make_async_copy(
            tbuf.at[b], probs_out.at[pl.ds(rows0(b), 8), pl.ds(_VMAIN, _TAIL)],
            sems_t)

    def start_in(b, slot):
        in_cp(b, slot).start()

        @pl.when(extra)
        def _():
            in_cp_x(b, slot).start()

    def wait_in(b, slot):
        in_cp(b, slot).wait()

        @pl.when(extra)
        def _():
            in_cp_x(b, slot).wait()

    def start_out(b, slot):
        out_cp(b, slot).start()

        @pl.when(extra)
        def _():
            out_cp_x(b, slot).start()

    def wait_out(b, slot):
        out_cp(b, slot).wait()

        @pl.when(extra)
        def _():
            out_cp_x(b, slot).wait()

    @pl.when(tail_here)
    def _():
        for b in range(_BANDS):
            t_in(b).start()
        for b in range(_BANDS):
            t_in(b).wait()
        for b in range(_BANDS):
            t_out(b).start()

    for j in range(_NBUF):
        start_in(j, j)

    cvals = [jnp.zeros((16,), jnp.float32) for _ in range(4)]
    ccols = [jnp.zeros((16,), jnp.int32) for _ in range(4)]

    for b in range(_BANDS):
        slot = b % _NBUF
        wait_in(b, slot)
        start_out(b, slot)

        vm = [jnp.full((16,), -jnp.inf, jnp.float32) for _ in range(8)]
        vg = [jnp.zeros((16,), jnp.int32) for _ in range(8)]

        def step(k, carry, slot=slot):
            acc = list(carry)
            for r in range(8):
                v = buf[slot, r, pl.ds(k * 16, 16)]
                m = v > acc[r]
                acc[r] = jnp.where(m, v, acc[r])
                acc[8 + r] = jnp.where(m, k, acc[8 + r])
            return tuple(acc)

        carry = tuple(vm + vg)
        carry = lax.fori_loop(0, _NVEC, step, carry)
        vm, vg = list(carry[:8]), list(carry[8:])

        moff_x = jnp.where(extra, 0.0, -jnp.inf)
        moff_t = jnp.where(tail_here, 0.0, -jnp.inf)
        for kk in range(8):
            k = _NVEC + kk
            for r in range(8):
                vx = buf[slot, r, pl.ds(k * 16, 16)] + moff_x
                m = vx > vm[r]
                vm[r] = jnp.where(m, vx, vm[r])
                vg[r] = jnp.where(m, k, vg[r])
                if kk < 2:
                    vt = tbuf[b, r, pl.ds(kk * 16, 16)] + moff_t
                    mt = vt > vm[r]
                    vm[r] = jnp.where(mt, vt, vm[r])
                    vg[r] = jnp.where(mt, k, vg[r])

        for r in range(8):
            gmax = _allmax(vm[r], lane)
            col = vg[r] * 16 + lane + start
            cand = jnp.where(vm[r] == gmax, col, _BIG)
            gcol = _allmin(cand, lane)
            tgt, ln = divmod(b * 8 + r, 16)
            cvals[tgt] = jnp.where(lane == ln, gmax, cvals[tgt])
            ccols[tgt] = jnp.where(lane == ln, gcol, ccols[tgt])

        nxt = b + _NBUF
        if nxt < _BANDS:
            wait_out(b, slot)
            start_in(nxt, slot)

    for b in range(_BANDS - _NBUF, _BANDS):
        wait_out(b, b % _NBUF)

    for i in range(4):
        valc[pl.ds(i * 16, 16)] = cvals[i]
        colc[pl.ds(i * 16, 16)] = ccols[i]

    pltpu.sync_copy(valc, sval.at[pl.ds(s * 64, 64)])
    pltpu.sync_copy(colc, scol.at[pl.ds(s * 64, 64)])
    plsc.subcore_barrier()
    pltpu.sync_copy(sval, mval)
    pltpu.sync_copy(scol, mcol)

    vs = [mval[pl.ds(sp * 64 + 4 * s, 16)] for sp in range(_NS)]
    cs = [mcol[pl.ds(sp * 64 + 4 * s, 16)] for sp in range(_NS)]
    mx = vs[0]
    for sp in range(1, _NS):
        mx = jnp.maximum(mx, vs[sp])
    ids_vec = jnp.full((16,), _BIG, jnp.int32)
    for sp in range(_NS):
        ids_vec = jnp.minimum(
            ids_vec, jnp.where(vs[sp] == mx, cs[sp], _BIG))

    @pl.when(tail_here)
    def _():
        for b in range(_BANDS):
            t_out(b).wait()

    idsv[...] = ids_vec
    pltpu.sync_copy(idsv, ids_out.at[c * _NS + s])


@functools.partial(jax.jit, static_argnames=())
def _sc_call(logits):
    f = pl.kernel(
        _sc_body,
        mesh=plsc.VectorSubcoreMesh(core_axis_name="c", subcore_axis_name="s"),
        out_type=[
            jax.ShapeDtypeStruct((2 * _NS, 16), jnp.int32),
            jax.ShapeDtypeStruct((_B, _V), jnp.float32),
        ],
        scratch_types=[
            pltpu.VMEM((_NBUF, 8, 49 * 128), jnp.float32),
            pltpu.VMEM((_BANDS, 8, _TAIL), jnp.float32),
            pltpu.VMEM((64,), jnp.float32),
            pltpu.VMEM((64,), jnp.int32),
            pltpu.VMEM((16,), jnp.int32),
            pltpu.VMEM_SHARED((_NS * 64 + 16,), jnp.float32),
            pltpu.VMEM_SHARED((_NS * 64 + 16,), jnp.int32),
            pltpu.VMEM((_NS * 64 + 16,), jnp.float32),
            pltpu.VMEM((_NS * 64 + 16,), jnp.int32),
            pltpu.SemaphoreType.DMA((_NBUF,)),
            pltpu.SemaphoreType.DMA((_NBUF,)),
            pltpu.SemaphoreType.DMA,
        ],
    )
    return f(logits)


def kernel(logits):
    ids2d, probs = _sc_call(logits)
    ids = ids2d[:, :4].reshape(_B)
    return ids, probs
